# baseline (speedup 1.0000x reference)
import functools

import jax
import jax.numpy as jnp
from jax import lax
from jax.experimental import pallas as pl
from jax.experimental.pallas import tpu as pltpu, tpu_sc as plsc


def _make_gather(V, D, B):
    info = plsc.get_sparse_core_info()
    NC, NS = info.num_cores, info.num_subcores
    NW = NC * NS
    assert B % (8 * NW) == 0
    b_per_w = B // NW
    mesh = plsc.VectorSubcoreMesh(core_axis_name="c", subcore_axis_name="s")

    @functools.partial(
        pl.kernel,
        mesh=mesh,
        out_type=jax.ShapeDtypeStruct((B, D), jnp.float32),
        compiler_params=pltpu.CompilerParams(use_tc_tiling_on_sc=False),
        scratch_types=[
            pltpu.VMEM((b_per_w,), jnp.int32),
            pltpu.VMEM((b_per_w, D), jnp.float32),
            pltpu.SemaphoreType.DMA,
        ],
    )
    def gather_k(table_hbm, idx_hbm, out_hbm, idx_v, rows_v, sem):
        wid = lax.axis_index("s") * NC + lax.axis_index("c")
        base = wid * b_per_w
        pltpu.sync_copy(idx_hbm.at[pl.ds(base, b_per_w)], idx_v)
        pltpu.async_copy(table_hbm.at[idx_v], rows_v, sem).wait()
        pltpu.sync_copy(rows_v, out_hbm.at[pl.ds(base, b_per_w)])

    return gather_k


def kernel(data, channelindex):
    V, D = data.shape
    (B,) = channelindex.shape
    idx = channelindex.astype(jnp.int32)
    return _make_gather(V, D, B)(data, idx)


# speedup vs baseline: 1.3283x; 1.3283x over previous
import functools

import jax
import jax.numpy as jnp
from jax import lax
from jax.experimental import pallas as pl
from jax.experimental.pallas import tpu as pltpu, tpu_sc as plsc


def _make_gather(V, D, B):
    info = plsc.get_sparse_core_info()
    NC, NS = info.num_cores, info.num_subcores
    NW = NC * NS
    assert B % (8 * NW) == 0
    b_per_w = B // NW
    K = 16
    mesh = plsc.VectorSubcoreMesh(core_axis_name="c", subcore_axis_name="s")

    @functools.partial(
        pl.kernel,
        mesh=mesh,
        out_type=jax.ShapeDtypeStruct((B, D), jnp.float32),
        scratch_types=[
            pltpu.VMEM((b_per_w,), jnp.int32),
            pltpu.VMEM((b_per_w, D), jnp.float32),
            pltpu.SemaphoreType.DMA,
        ],
    )
    def gather_k(table_hbm, idx_hbm, out_hbm, idx_v, rows_v, sem):
        wid = lax.axis_index("s") * NC + lax.axis_index("c")
        base = wid * b_per_w
        pltpu.sync_copy(idx_hbm.at[pl.ds(base, b_per_w)], idx_v)

        def issue_chunk(c):
            v = idx_v[pl.ds(c * 16, 16)]
            for l in range(16):
                pltpu.async_copy(
                    table_hbm.at[pl.ds(v[l], 1)],
                    rows_v.at[pl.ds(c * 16 + l, 1)],
                    sem,
                )

        def drain_chunk():
            for _ in range(16):
                pltpu.make_async_copy(
                    table_hbm.at[pl.ds(0, 1)], rows_v.at[pl.ds(0, 1)], sem
                ).wait()

        n_chunks = b_per_w // 16

        @pl.loop(0, n_chunks)
        def _main(c):
            issue_chunk(c)

            @pl.when(c > 0)
            def _():
                drain_chunk()

        drain_chunk()

        pltpu.sync_copy(rows_v, out_hbm.at[pl.ds(base, b_per_w)])

    return gather_k


def kernel(data, channelindex):
    V, D = data.shape
    (B,) = channelindex.shape
    idx = channelindex.astype(jnp.int32)
    return _make_gather(V, D, B)(data, idx)


# speedup vs baseline: 1.3927x; 1.0484x over previous
import functools

import jax
import jax.numpy as jnp
from jax import lax
from jax.experimental import pallas as pl
from jax.experimental.pallas import tpu as pltpu, tpu_sc as plsc


def _make_gather(V, D, B):
    info = plsc.get_sparse_core_info()
    NC, NS = info.num_cores, info.num_subcores
    NW = NC * NS
    assert B % (8 * NW) == 0
    b_per_w = B // NW
    mesh = plsc.VectorSubcoreMesh(core_axis_name="c", subcore_axis_name="s")

    @functools.partial(
        pl.kernel,
        mesh=mesh,
        out_type=jax.ShapeDtypeStruct((B, D), jnp.float32),
        scratch_types=[
            pltpu.VMEM((b_per_w,), jnp.int32),
            pltpu.VMEM((b_per_w, D), jnp.float32),
            pltpu.SemaphoreType.DMA,
        ],
    )
    def gather_k(table_hbm, idx_hbm, out_hbm, idx_v, rows_v, sem):
        wid = lax.axis_index("s") * NC + lax.axis_index("c")
        base = wid * b_per_w
        pltpu.sync_copy(idx_hbm.at[pl.ds(base, b_per_w)], idx_v)

        def issue_chunk(c):
            v = idx_v[pl.ds(c * 16, 16)]
            for l in range(16):
                pltpu.async_copy(
                    table_hbm.at[pl.ds(v[l], 1)],
                    rows_v.at[pl.ds(c * 16 + l, 1)],
                    sem,
                )

        def drain_chunk():
            pltpu.make_async_copy(
                table_hbm.at[pl.ds(0, 16)], rows_v.at[pl.ds(0, 16)], sem
            ).wait()

        n_chunks = b_per_w // 16

        @pl.loop(0, n_chunks)
        def _main(c):
            issue_chunk(c)

            @pl.when(c >= 2)
            def _():
                drain_chunk()

        drain_chunk()
        drain_chunk()

        pltpu.sync_copy(rows_v, out_hbm.at[pl.ds(base, b_per_w)])

    return gather_k


def kernel(data, channelindex):
    V, D = data.shape
    (B,) = channelindex.shape
    idx = channelindex.astype(jnp.int32)
    return _make_gather(V, D, B)(data, idx)


# speedup vs baseline: 1.4462x; 1.0385x over previous
import functools

import jax
import jax.numpy as jnp
from jax import lax
from jax.experimental import pallas as pl
from jax.experimental.pallas import tpu as pltpu, tpu_sc as plsc


def _make_gather(V, D, B):
    info = plsc.get_sparse_core_info()
    NC, NS = info.num_cores, info.num_subcores
    NW = NC * NS
    assert B % (8 * NW) == 0
    b_per_w = B // NW
    mesh = plsc.VectorSubcoreMesh(core_axis_name="c", subcore_axis_name="s")

    @functools.partial(
        pl.kernel,
        mesh=mesh,
        out_type=jax.ShapeDtypeStruct((B, D), jnp.float32),
        scratch_types=[
            pltpu.VMEM((b_per_w,), jnp.int32),
            pltpu.VMEM((b_per_w, D), jnp.float32),
            pltpu.SemaphoreType.DMA,
            pltpu.SemaphoreType.DMA,
        ],
    )
    def gather_k(table_hbm, idx_hbm, out_hbm, idx_v, rows_v, sem0, sem1):
        wid = lax.axis_index("s") * NC + lax.axis_index("c")
        base = wid * b_per_w
        pltpu.sync_copy(idx_hbm.at[pl.ds(base, b_per_w)], idx_v)
        sems = (sem0, sem1)

        def issue_chunk(c, sem):
            v = idx_v[pl.ds(c * 16, 16)]
            for l in range(16):
                pltpu.async_copy(
                    table_hbm.at[pl.ds(v[l], 1)],
                    rows_v.at[pl.ds(c * 16 + l, 1)],
                    sem,
                )

        def drain_chunk(sem):
            pltpu.make_async_copy(
                table_hbm.at[pl.ds(0, 16)], rows_v.at[pl.ds(0, 16)], sem
            ).wait()

        n_chunks = b_per_w // 16
        LAG = 4

        @pl.loop(0, n_chunks // 2)
        def _main(h):
            c = h * 2
            issue_chunk(c, sems[0])
            issue_chunk(c + 1, sems[1])

            @pl.when(c >= LAG)
            def _():
                drain_chunk(sems[0])
                drain_chunk(sems[1])

        for _ in range(LAG // 2):
            drain_chunk(sems[0])
            drain_chunk(sems[1])

        pltpu.sync_copy(rows_v, out_hbm.at[pl.ds(base, b_per_w)])

    return gather_k


def kernel(data, channelindex):
    V, D = data.shape
    (B,) = channelindex.shape
    idx = channelindex.astype(jnp.int32)
    return _make_gather(V, D, B)(data, idx)
